# baseline (speedup 1.0000x reference)
import functools

import jax
import jax.numpy as jnp
from jax import lax
from jax.experimental import pallas as pl
from jax.experimental.pallas import tpu as pltpu
from jax.experimental.pallas import tpu_sc as plsc

BATCH = 16384
HIDDEN = 64
LANES = 128
NUM_WORKERS = 32
RPW = BATCH // NUM_WORKERS
NCH = RPW // LANES
NBUF = 6


def _gather_kernel(tabT, lab_hbm, pos_hbm, out_hbm,
                   labv, posv, rectring, stage, rsv, rsem, osem):
    wid = lax.axis_index("s") * 2 + lax.axis_index("c")
    pltpu.sync_copy(lab_hbm.at[wid], labv)
    pltpu.sync_copy(pos_hbm.at[wid], posv)

    iot = lax.iota(jnp.int32, 16)

    def lab_splat(i):
        grp = (i >> 4) & 7
        v = labv[i >> 7, pl.ds(pl.multiple_of(grp * 16, 16), 16)]
        idx = jnp.broadcast_to(i & 15, (16,))
        return lax.gather(
            v, idx[:, None],
            dimension_numbers=lax.GatherDimensionNumbers(
                offset_dims=(), collapsed_slice_dims=(0,),
                start_index_map=(0,)),
            slice_sizes=(1,),
            mode=lax.GatherScatterMode.PROMISE_IN_BOUNDS)

    def lab_at(i):
        return jnp.sum(jnp.where(iot == 0, lab_splat(i), 0))

    def start_rect_tc(tc, b):
        start = pl.multiple_of(tc * LANES, LANES)
        return pltpu.async_copy(
            tabT.at[:, pl.ds(start, LANES)], rectring.at[b], rsem.at[b])

    def start_rect(i_first, b):
        return start_rect_tc(lab_at(i_first) >> 7, b)

    def rd(i, carry):
        cnt, prev = carry
        tc = lax.shift_right_logical(lab_splat(i), 7)
        new = jnp.any(tc != prev)

        @pl.when(new)
        def _():
            rsv[cnt] = i

            @pl.when(cnt < NBUF)
            def _():
                start_rect_tc(jnp.sum(jnp.where(iot == 0, tc, 0)), cnt)

        return cnt + new.astype(jnp.int32), tc

    nruns, _ = lax.fori_loop(
        0, RPW, rd, (jnp.int32(0), jnp.full((16,), -1, jnp.int32)))
    rsv[nruns] = jnp.int32(RPW)

    hvecs = [iot + j * 16 for j in range(HIDDEN // 16)]

    def run_body(r, _):
        p = lax.rem(r, jnp.int32(NBUF))
        pltpu.make_async_copy(
            tabT.at[:, pl.ds(0, LANES)], rectring.at[p], rsem.at[p]).wait()

        def ext(i, _):
            lane = lab_splat(i) & (LANES - 1)
            for j in range(HIDDEN // 16):
                g = plsc.load_gather(rectring.at[p], [hvecs[j], lane])
                stage[i, pl.ds(j * 16, 16)] = g
            return 0

        lax.fori_loop(rsv[r], rsv[r + 1], ext, 0)

        @pl.when(r + NBUF < nruns)
        def _():
            start_rect(rsv[r + NBUF], p)

        return 0

    lax.fori_loop(0, nruns, run_body, 0)

    cps = []
    for c in range(NCH):
        for k in range(LANES // 16):
            pv = posv[c, pl.ds(k * 16, 16)]
            cps.append(pltpu.async_copy(
                stage.at[pl.ds(c * LANES + k * 16, 16)], out_hbm.at[pv], osem))
    for cp in cps:
        cp.wait()


@jax.jit
def _embed(lab3, pos3, tabT):
    mesh = plsc.VectorSubcoreMesh(core_axis_name="c", subcore_axis_name="s")
    run = functools.partial(
        pl.kernel,
        mesh=mesh,
        out_type=jax.ShapeDtypeStruct((BATCH, LANES), jnp.float32),
        scratch_types=[
            pltpu.VMEM((NCH, LANES), jnp.int32),
            pltpu.VMEM((NCH, LANES), jnp.int32),
            pltpu.VMEM((NBUF, HIDDEN, LANES), jnp.float32),
            pltpu.VMEM((RPW, LANES), jnp.float32),
            pltpu.SMEM((RPW + 32, ), jnp.int32),
            pltpu.SemaphoreType.DMA((NBUF,)),
            pltpu.SemaphoreType.DMA,
        ],
        compiler_params=pltpu.CompilerParams(
            disable_bounds_checks=True, needs_layout_passes=False),
    )(_gather_kernel)
    return run(tabT, lab3, pos3)


def kernel(labels, train, table):
    del train
    order = lax.iota(jnp.int32, BATCH)
    slab, order = lax.sort((labels.astype(jnp.int32), order), num_keys=1)
    out_p = _embed(slab.reshape(NUM_WORKERS, NCH, LANES),
                   order.reshape(NUM_WORKERS, NCH, LANES),
                   table.T)
    return out_p[:, :HIDDEN]


# speedup vs baseline: 1.0679x; 1.0679x over previous
import functools

import jax
import jax.numpy as jnp
from jax import lax
from jax.experimental import pallas as pl
from jax.experimental.pallas import tpu as pltpu
from jax.experimental.pallas import tpu_sc as plsc

BATCH = 16384
HIDDEN = 64
LANES = 128
NUM_WORKERS = 32
RPW = BATCH // NUM_WORKERS
NCH = RPW // LANES
NBUF = 6


def _gather_kernel(tabT, lab_hbm, pos_hbm, out_hbm,
                   labv, posv, rectring, stage, rsv, rsem, osem):
    wid = lax.axis_index("s") * 2 + lax.axis_index("c")
    pltpu.sync_copy(lab_hbm.at[wid], labv)
    pltpu.sync_copy(pos_hbm.at[wid], posv)

    iot = lax.iota(jnp.int32, 16)

    def lab_splat(i):
        grp = (i >> 4) & 7
        v = labv[i >> 7, pl.ds(pl.multiple_of(grp * 16, 16), 16)]
        idx = jnp.broadcast_to(i & 15, (16,))
        return lax.gather(
            v, idx[:, None],
            dimension_numbers=lax.GatherDimensionNumbers(
                offset_dims=(), collapsed_slice_dims=(0,),
                start_index_map=(0,)),
            slice_sizes=(1,),
            mode=lax.GatherScatterMode.PROMISE_IN_BOUNDS)

    def lab_at(i):
        return jnp.sum(jnp.where(iot == 0, lab_splat(i), 0))

    def start_rect_tc(tc, b):
        start = pl.multiple_of(tc * LANES, LANES)
        return pltpu.async_copy(
            tabT.at[:, pl.ds(start, LANES)], rectring.at[b], rsem.at[b])

    def start_rect(i_first, b):
        return start_rect_tc(lab_at(i_first) >> 7, b)

    def rd(i, carry):
        cnt, prev = carry
        tc = lab_at(i) >> 7
        new = tc != prev

        @pl.when(new)
        def _():
            rsv[cnt] = i

            @pl.when(cnt < NBUF)
            def _():
                start_rect_tc(tc, cnt)

        return cnt + new.astype(jnp.int32), tc

    nruns, _ = lax.fori_loop(0, RPW, rd, (jnp.int32(0), jnp.int32(-1)))
    rsv[nruns] = jnp.int32(RPW)

    hvecs = [iot + j * 16 for j in range(HIDDEN // 16)]

    def run_body(r, _):
        p = lax.rem(r, jnp.int32(NBUF))
        pltpu.make_async_copy(
            tabT.at[:, pl.ds(0, LANES)], rectring.at[p], rsem.at[p]).wait()

        def ext(i, _):
            lane = lab_splat(i) & (LANES - 1)
            for j in range(HIDDEN // 16):
                g = plsc.load_gather(rectring.at[p], [hvecs[j], lane])
                stage[i, pl.ds(j * 16, 16)] = g
            return 0

        lax.fori_loop(rsv[r], rsv[r + 1], ext, 0)

        @pl.when(r + NBUF < nruns)
        def _():
            start_rect(rsv[r + NBUF], p)

        return 0

    lax.fori_loop(0, nruns, run_body, 0)

    cps = []
    for c in range(NCH):
        for k in range(LANES // 16):
            pv = posv[c, pl.ds(k * 16, 16)]
            cps.append(pltpu.async_copy(
                stage.at[pl.ds(c * LANES + k * 16, 16)], out_hbm.at[pv], osem))
    for cp in cps:
        cp.wait()


@jax.jit
def _embed(lab3, pos3, tabT):
    mesh = plsc.VectorSubcoreMesh(core_axis_name="c", subcore_axis_name="s")
    run = functools.partial(
        pl.kernel,
        mesh=mesh,
        out_type=jax.ShapeDtypeStruct((BATCH, LANES), jnp.float32),
        scratch_types=[
            pltpu.VMEM((NCH, LANES), jnp.int32),
            pltpu.VMEM((NCH, LANES), jnp.int32),
            pltpu.VMEM((NBUF, HIDDEN, LANES), jnp.float32),
            pltpu.VMEM((RPW, LANES), jnp.float32),
            pltpu.SMEM((RPW + 32, ), jnp.int32),
            pltpu.SemaphoreType.DMA((NBUF,)),
            pltpu.SemaphoreType.DMA,
        ],
        compiler_params=pltpu.CompilerParams(
            disable_bounds_checks=True, needs_layout_passes=False),
    )(_gather_kernel)
    return run(tabT, lab3, pos3)


def kernel(labels, train, table):
    del train
    order = lax.iota(jnp.int32, BATCH)
    slab, order = lax.sort((labels.astype(jnp.int32), order), num_keys=1)
    out_p = _embed(slab.reshape(NUM_WORKERS, NCH, LANES),
                   order.reshape(NUM_WORKERS, NCH, LANES),
                   table.T)
    return out_p[:, :HIDDEN]
